# Initial kernel scaffold; baseline (speedup 1.0000x reference)
import jax
import jax.numpy as jnp
from jax.experimental import pallas as pl


def kernel(user_id, history, top_genres, avg_rating, activity, user_table, item_table, genre_table, W_cont, b_cont, W1, b1, W2, b2):
    raise NotImplementedError("write your pallas kernel here")


# speedup vs baseline: 4.4632x; 4.4632x over previous
import functools

import jax
import jax.numpy as jnp
from jax import lax
from jax.experimental import pallas as pl
from jax.experimental.pallas import tpu as pltpu
from jax.experimental.pallas import tpu_sc as plsc

B = 4096
HIST = 50
NG = 8
GENRE_VOCAB = 21
D = 64

NC = 2
NS = 16
NW = NC * NS
BPW = B // NW


@functools.partial(
    pl.kernel,
    out_type=(
        jax.ShapeDtypeStruct((B, D), jnp.float32),
        jax.ShapeDtypeStruct((B, D), jnp.float32),
    ),
    mesh=plsc.VectorSubcoreMesh(
        core_axis_name="c", subcore_axis_name="s",
        num_cores=NC, num_subcores=NS),
    scratch_types=[
        pltpu.VMEM((BPW,), jnp.int32),
        pltpu.VMEM((BPW, D), jnp.float32),
        pltpu.VMEM((BPW, HIST), jnp.int32),
        pltpu.VMEM((HIST, D), jnp.float32),
        pltpu.VMEM((BPW, D), jnp.float32),
        pltpu.SemaphoreType.DMA,
    ],
    compiler_params=pltpu.CompilerParams(use_tc_tiling_on_sc=False),
)
def _sc_gather_pool(uid_hbm, hist_hbm, utab_hbm, itab_hbm,
                    u_out, hsum_out,
                    uid_v, urows_v, idx_v, rows_v, hsum_v, sem):
    w = lax.axis_index("s") * NC + lax.axis_index("c")
    base = w * BPW

    pltpu.sync_copy(uid_hbm.at[pl.ds(base, BPW)], uid_v)
    pltpu.async_copy(utab_hbm.at[uid_v], urows_v, sem).wait()
    pltpu.sync_copy(urows_v, u_out.at[pl.ds(base, BPW), :])

    pltpu.sync_copy(hist_hbm.at[w], idx_v)

    def row_body(r, carry):
        pltpu.async_copy(itab_hbm.at[idx_v.at[r]], rows_v, sem).wait()

        def acc_body(j, acc):
            return tuple(acc[c] + rows_v[j, pl.ds(16 * c, 16)]
                         for c in range(4))

        z = jnp.zeros((16,), jnp.float32)
        acc = lax.fori_loop(0, HIST, acc_body, (z, z, z, z))
        for c in range(4):
            hsum_v[r, pl.ds(16 * c, 16)] = acc[c]
        return carry

    lax.fori_loop(0, BPW, row_body, 0)
    pltpu.sync_copy(hsum_v, hsum_out.at[pl.ds(base, BPW), :])


def _tc_mlp(u_ref, hs_ref, hist_ref, tg_ref, cont_ref, gtab_ref,
            wc_ref, bc_ref, w1_ref, b1_ref, w2_ref, b2_ref, o_ref):
    hist = hist_ref[...]
    h_cnt = jnp.sum((hist > 0).astype(jnp.float32), axis=1, keepdims=True)
    hist_pool = hs_ref[...] / (h_cnt + 1e-8)

    tg = tg_ref[...]
    gtab = gtab_ref[...]
    g_cnt = jnp.sum((tg > 0).astype(jnp.float32), axis=1, keepdims=True)
    g_sum = jnp.zeros_like(hist_pool)
    for v in range(1, GENRE_VOCAB):
        cv = jnp.sum((tg == v).astype(jnp.float32), axis=1, keepdims=True)
        g_sum = g_sum + cv * gtab[v:v + 1, :]
    g_pool = g_sum / (g_cnt + 1e-8)

    cont = cont_ref[...]
    wc = wc_ref[...]
    cont_emb = jnp.maximum(
        cont[:, 0:1] * wc[0:1, :] + cont[:, 1:2] * wc[1:2, :] + bc_ref[...],
        0.0)

    w1 = w1_ref[...]
    f32 = jnp.float32
    h = (jnp.dot(u_ref[...], w1[0:64], preferred_element_type=f32)
         + jnp.dot(hist_pool, w1[64:128], preferred_element_type=f32)
         + jnp.dot(g_pool, w1[128:192], preferred_element_type=f32)
         + jnp.dot(cont_emb, w1[192:256], preferred_element_type=f32)
         + b1_ref[...])
    h = jnp.maximum(h, 0.0)
    out = jnp.dot(h, w2_ref[...], preferred_element_type=f32) + b2_ref[...]
    nrm = jnp.sqrt(jnp.sum(out * out, axis=1, keepdims=True))
    o_ref[...] = out / jnp.maximum(nrm, 1e-12)


def kernel(user_id, history, top_genres, avg_rating, activity,
           user_table, item_table, genre_table,
           W_cont, b_cont, W1, b1, W2, b2):
    uid = user_id.astype(jnp.int32)
    hist = history.astype(jnp.int32)
    hist3 = hist.reshape(NW, BPW, HIST)

    u_emb, hsum = _sc_gather_pool(uid, hist3, user_table, item_table)

    cont = jnp.stack([avg_rating, activity], axis=1)

    bb = 512
    grid = (B // bb,)
    full = lambda shape: pl.BlockSpec(shape, lambda i: (0, 0))
    blk = lambda shape: pl.BlockSpec(shape, lambda i: (i, 0))

    out = pl.pallas_call(
        _tc_mlp,
        grid=grid,
        in_specs=[
            blk((bb, D)),
            blk((bb, D)),
            blk((bb, HIST)),
            blk((bb, NG)),
            blk((bb, 2)),
            full((GENRE_VOCAB, D)),
            full((2, D)),
            full((1, D)),
            full((4 * D, 128)),
            full((1, 128)),
            full((128, D)),
            full((1, D)),
        ],
        out_specs=blk((bb, D)),
        out_shape=jax.ShapeDtypeStruct((B, D), jnp.float32),
    )(u_emb, hsum, hist, top_genres.astype(jnp.int32), cont, genre_table,
      W_cont, b_cont.reshape(1, D), W1, b1.reshape(1, 128), W2,
      b2.reshape(1, D))
    return out


# speedup vs baseline: 5.6667x; 1.2696x over previous
import functools

import jax
import jax.numpy as jnp
from jax import lax
from jax.experimental import pallas as pl
from jax.experimental.pallas import tpu as pltpu
from jax.experimental.pallas import tpu_sc as plsc

B = 4096
HIST = 50
NG = 8
GENRE_VOCAB = 21
D = 64

NC = 2
NS = 16
NW = NC * NS
BPW = B // NW


@functools.partial(
    pl.kernel,
    out_type=(
        jax.ShapeDtypeStruct((B, D), jnp.float32),
        jax.ShapeDtypeStruct((B, D), jnp.float32),
    ),
    mesh=plsc.VectorSubcoreMesh(
        core_axis_name="c", subcore_axis_name="s",
        num_cores=NC, num_subcores=NS),
    scratch_types=[
        pltpu.VMEM((BPW,), jnp.int32),
        pltpu.VMEM((BPW, D), jnp.float32),
        pltpu.VMEM((BPW, HIST), jnp.int32),
        pltpu.VMEM((HIST, D), jnp.float32),
        pltpu.VMEM((HIST, D), jnp.float32),
        pltpu.VMEM((BPW, D), jnp.float32),
        pltpu.SemaphoreType.DMA,
        pltpu.SemaphoreType.DMA,
        pltpu.SemaphoreType.DMA,
    ],
    compiler_params=pltpu.CompilerParams(use_tc_tiling_on_sc=False),
)
def _sc_gather_pool(uid_hbm, hist_hbm, utab_hbm, itab_hbm,
                    u_out, hsum_out,
                    uid_v, urows_v, idx_v, rows_a, rows_b, hsum_v,
                    sem_u, sem_a, sem_b):
    w = lax.axis_index("s") * NC + lax.axis_index("c")
    base = w * BPW

    pltpu.sync_copy(uid_hbm.at[pl.ds(base, BPW)], uid_v)
    pltpu.async_copy(utab_hbm.at[uid_v], urows_v, sem_u)

    pltpu.sync_copy(hist_hbm.at[w], idx_v)

    def _accum(rows, r):
        def acc_body(j, acc):
            return tuple(acc[c] + rows[j, pl.ds(16 * c, 16)]
                         for c in range(4))

        z = jnp.zeros((16,), jnp.float32)
        acc = lax.fori_loop(0, HIST, acc_body, (z, z, z, z), unroll=5)
        for c in range(4):
            hsum_v[r, pl.ds(16 * c, 16)] = acc[c]

    pltpu.async_copy(itab_hbm.at[idx_v.at[0]], rows_a, sem_a)

    def pair_body(g, carry):
        r0 = 2 * g
        pltpu.async_copy(itab_hbm.at[idx_v.at[r0 + 1]], rows_b, sem_b)
        pltpu.make_async_copy(itab_hbm.at[idx_v.at[r0]], rows_a, sem_a).wait()
        _accum(rows_a, r0)

        @pl.when(r0 + 2 < BPW)
        def _():
            pltpu.async_copy(itab_hbm.at[idx_v.at[r0 + 2]], rows_a, sem_a)

        pltpu.make_async_copy(
            itab_hbm.at[idx_v.at[r0 + 1]], rows_b, sem_b).wait()
        _accum(rows_b, r0 + 1)
        return carry

    lax.fori_loop(0, BPW // 2, pair_body, 0)
    pltpu.sync_copy(hsum_v, hsum_out.at[pl.ds(base, BPW), :])

    pltpu.make_async_copy(utab_hbm.at[uid_v], urows_v, sem_u).wait()
    pltpu.sync_copy(urows_v, u_out.at[pl.ds(base, BPW), :])


def _tc_mlp(u_ref, hs_ref, hist_ref, tg_ref, cont_ref, gtab_ref,
            wc_ref, bc_ref, w1_ref, b1_ref, w2_ref, b2_ref, o_ref):
    hist = hist_ref[...]
    h_cnt = jnp.sum((hist > 0).astype(jnp.float32), axis=1, keepdims=True)
    hist_pool = hs_ref[...] / (h_cnt + 1e-8)

    tg = tg_ref[...]
    gtab = gtab_ref[...]
    g_cnt = jnp.sum((tg > 0).astype(jnp.float32), axis=1, keepdims=True)
    g_sum = jnp.zeros_like(hist_pool)
    for v in range(1, GENRE_VOCAB):
        cv = jnp.sum((tg == v).astype(jnp.float32), axis=1, keepdims=True)
        g_sum = g_sum + cv * gtab[v:v + 1, :]
    g_pool = g_sum / (g_cnt + 1e-8)

    cont = cont_ref[...]
    wc = wc_ref[...]
    cont_emb = jnp.maximum(
        cont[:, 0:1] * wc[0:1, :] + cont[:, 1:2] * wc[1:2, :] + bc_ref[...],
        0.0)

    w1 = w1_ref[...]
    f32 = jnp.float32
    h = (jnp.dot(u_ref[...], w1[0:64], preferred_element_type=f32)
         + jnp.dot(hist_pool, w1[64:128], preferred_element_type=f32)
         + jnp.dot(g_pool, w1[128:192], preferred_element_type=f32)
         + jnp.dot(cont_emb, w1[192:256], preferred_element_type=f32)
         + b1_ref[...])
    h = jnp.maximum(h, 0.0)
    out = jnp.dot(h, w2_ref[...], preferred_element_type=f32) + b2_ref[...]
    nrm = jnp.sqrt(jnp.sum(out * out, axis=1, keepdims=True))
    o_ref[...] = out / jnp.maximum(nrm, 1e-12)


def kernel(user_id, history, top_genres, avg_rating, activity,
           user_table, item_table, genre_table,
           W_cont, b_cont, W1, b1, W2, b2):
    uid = user_id.astype(jnp.int32)
    hist = history.astype(jnp.int32)
    hist3 = hist.reshape(NW, BPW, HIST)

    u_emb, hsum = _sc_gather_pool(uid, hist3, user_table, item_table)

    cont = jnp.stack([avg_rating, activity], axis=1)

    bb = 512
    grid = (B // bb,)
    full = lambda shape: pl.BlockSpec(shape, lambda i: (0, 0))
    blk = lambda shape: pl.BlockSpec(shape, lambda i: (i, 0))

    out = pl.pallas_call(
        _tc_mlp,
        grid=grid,
        in_specs=[
            blk((bb, D)),
            blk((bb, D)),
            blk((bb, HIST)),
            blk((bb, NG)),
            blk((bb, 2)),
            full((GENRE_VOCAB, D)),
            full((2, D)),
            full((1, D)),
            full((4 * D, 128)),
            full((1, 128)),
            full((128, D)),
            full((1, D)),
        ],
        out_specs=blk((bb, D)),
        out_shape=jax.ShapeDtypeStruct((B, D), jnp.float32),
    )(u_emb, hsum, hist, top_genres.astype(jnp.int32), cont, genre_table,
      W_cont, b_cont.reshape(1, D), W1, b1.reshape(1, 128), W2,
      b2.reshape(1, D))
    return out


# speedup vs baseline: 5.9025x; 1.0416x over previous
import functools

import jax
import jax.numpy as jnp
from jax import lax
from jax.experimental import pallas as pl
from jax.experimental.pallas import tpu as pltpu
from jax.experimental.pallas import tpu_sc as plsc

B = 4096
HIST = 50
NG = 8
GENRE_VOCAB = 21
D = 64
VOCAB = 100001

NC = 2
NS = 16
NW = NC * NS
BPW = B // NW

TBLK = 1024
M = 98 * TBLK


def _tc_transpose(tin_ref, o_ref):
    o_ref[:, 0:D] = tin_ref[...].T


def _transpose_table(table_t):
    return pl.pallas_call(
        _tc_transpose,
        grid=(M // TBLK,),
        in_specs=[pl.BlockSpec((D, TBLK), lambda i: (0, i))],
        out_specs=pl.BlockSpec((TBLK, 128), lambda i: (i, 0)),
        out_shape=jax.ShapeDtypeStruct((M, 128), jnp.float32),
    )(table_t)


@functools.partial(
    pl.kernel,
    out_type=jax.ShapeDtypeStruct((B, 128), jnp.float32),
    mesh=plsc.VectorSubcoreMesh(
        core_axis_name="c", subcore_axis_name="s",
        num_cores=NC, num_subcores=NS),
    scratch_types=[
        pltpu.VMEM((BPW, HIST), jnp.int32),
        pltpu.VMEM((HIST, 128), jnp.float32),
        pltpu.VMEM((HIST, 128), jnp.float32),
        pltpu.VMEM((BPW, 128), jnp.float32),
        pltpu.SemaphoreType.DMA,
        pltpu.SemaphoreType.DMA,
    ],
    compiler_params=pltpu.CompilerParams(use_tc_tiling_on_sc=False),
)
def _sc_hist_pool(hist_hbm, itab_hbm, hsum_out,
                  idx_v, rows_a, rows_b, hsum_v, sem_a, sem_b):
    w = lax.axis_index("s") * NC + lax.axis_index("c")
    base = w * BPW

    pltpu.sync_copy(hist_hbm.at[w], idx_v)

    def _accum(rows, r):
        def acc_body(j, acc):
            return tuple(acc[c] + rows[j, pl.ds(16 * c, 16)]
                         for c in range(4))

        z = jnp.zeros((16,), jnp.float32)
        acc = lax.fori_loop(0, HIST, acc_body, (z, z, z, z), unroll=5)
        for c in range(4):
            hsum_v[r, pl.ds(16 * c, 16)] = acc[c]

    pltpu.async_copy(itab_hbm.at[idx_v.at[0]], rows_a, sem_a)

    def pair_body(g, carry):
        r0 = 2 * g
        pltpu.async_copy(itab_hbm.at[idx_v.at[r0 + 1]], rows_b, sem_b)
        pltpu.make_async_copy(itab_hbm.at[idx_v.at[r0]], rows_a, sem_a).wait()
        _accum(rows_a, r0)

        @pl.when(r0 + 2 < BPW)
        def _():
            pltpu.async_copy(itab_hbm.at[idx_v.at[r0 + 2]], rows_a, sem_a)

        pltpu.make_async_copy(
            itab_hbm.at[idx_v.at[r0 + 1]], rows_b, sem_b).wait()
        _accum(rows_b, r0 + 1)
        return carry

    lax.fori_loop(0, BPW // 2, pair_body, 0)
    pltpu.sync_copy(hsum_v, hsum_out.at[pl.ds(base, BPW), :])


@functools.partial(
    pl.kernel,
    out_type=jax.ShapeDtypeStruct((B, 128), jnp.float32),
    mesh=plsc.VectorSubcoreMesh(
        core_axis_name="c", subcore_axis_name="s",
        num_cores=NC, num_subcores=NS),
    scratch_types=[
        pltpu.VMEM((BPW,), jnp.int32),
        pltpu.VMEM((BPW, 128), jnp.float32),
        pltpu.SemaphoreType.DMA,
    ],
    compiler_params=pltpu.CompilerParams(use_tc_tiling_on_sc=False),
)
def _sc_user_gather(uid_hbm, utab_hbm, u_out, uid_v, urows_v, sem):
    w = lax.axis_index("s") * NC + lax.axis_index("c")
    base = w * BPW
    pltpu.sync_copy(uid_hbm.at[pl.ds(base, BPW)], uid_v)
    pltpu.async_copy(utab_hbm.at[uid_v], urows_v, sem).wait()
    pltpu.sync_copy(urows_v, u_out.at[pl.ds(base, BPW), :])


def _tc_mlp(u_ref, hs_ref, hist_ref, tg_ref, cont_ref, gtab_ref,
            wc_ref, bc_ref, w1_ref, b1_ref, w2_ref, b2_ref, o_ref):
    hist = hist_ref[...]
    h_cnt = jnp.sum((hist > 0).astype(jnp.float32), axis=1, keepdims=True)
    hist_pool = hs_ref[:, 0:D] / (h_cnt + 1e-8)

    tg = tg_ref[...]
    gtab = gtab_ref[...]
    g_cnt = jnp.sum((tg > 0).astype(jnp.float32), axis=1, keepdims=True)
    g_sum = jnp.zeros_like(hist_pool)
    for v in range(1, GENRE_VOCAB):
        cv = jnp.sum((tg == v).astype(jnp.float32), axis=1, keepdims=True)
        g_sum = g_sum + cv * gtab[v:v + 1, :]
    g_pool = g_sum / (g_cnt + 1e-8)

    cont = cont_ref[...]
    wc = wc_ref[...]
    cont_emb = jnp.maximum(
        cont[:, 0:1] * wc[0:1, :] + cont[:, 1:2] * wc[1:2, :] + bc_ref[...],
        0.0)

    w1 = w1_ref[...]
    f32 = jnp.float32
    h = (jnp.dot(u_ref[:, 0:D], w1[0:64], preferred_element_type=f32)
         + jnp.dot(hist_pool, w1[64:128], preferred_element_type=f32)
         + jnp.dot(g_pool, w1[128:192], preferred_element_type=f32)
         + jnp.dot(cont_emb, w1[192:256], preferred_element_type=f32)
         + b1_ref[...])
    h = jnp.maximum(h, 0.0)
    out = jnp.dot(h, w2_ref[...], preferred_element_type=f32) + b2_ref[...]
    nrm = jnp.sqrt(jnp.sum(out * out, axis=1, keepdims=True))
    o_ref[...] = out / jnp.maximum(nrm, 1e-12)


def kernel(user_id, history, top_genres, avg_rating, activity,
           user_table, item_table, genre_table,
           W_cont, b_cont, W1, b1, W2, b2):
    uid = user_id.astype(jnp.int32)
    hist = history.astype(jnp.int32)
    hist3 = hist.reshape(NW, BPW, HIST)

    itabM = _transpose_table(item_table.T)
    hsum = _sc_hist_pool(hist3, itabM)
    utabM = _transpose_table(user_table.T)
    u_emb = _sc_user_gather(uid, utabM)

    cont = jnp.stack([avg_rating, activity], axis=1)

    bb = 512
    grid = (B // bb,)
    full = lambda shape: pl.BlockSpec(shape, lambda i: (0, 0))
    blk = lambda shape: pl.BlockSpec(shape, lambda i: (i, 0))

    out = pl.pallas_call(
        _tc_mlp,
        grid=grid,
        in_specs=[
            blk((bb, 128)),
            blk((bb, 128)),
            blk((bb, HIST)),
            blk((bb, NG)),
            blk((bb, 2)),
            full((GENRE_VOCAB, D)),
            full((2, D)),
            full((1, D)),
            full((4 * D, 128)),
            full((1, 128)),
            full((128, D)),
            full((1, D)),
        ],
        out_specs=blk((bb, D)),
        out_shape=jax.ShapeDtypeStruct((B, D), jnp.float32),
    )(u_emb, hsum, hist, top_genres.astype(jnp.int32), cont, genre_table,
      W_cont, b_cont.reshape(1, D), W1, b1.reshape(1, 128), W2,
      b2.reshape(1, D))
    return out


# speedup vs baseline: 7.9908x; 1.3538x over previous
import functools

import jax
import jax.numpy as jnp
from jax import lax
from jax.experimental import pallas as pl
from jax.experimental.pallas import tpu as pltpu
from jax.experimental.pallas import tpu_sc as plsc

B = 4096
HIST = 50
NG = 8
GENRE_VOCAB = 21
D = 64
VOCAB = 100001

NC = 2
NS = 16
NW = NC * NS
BPW = B // NW

TBLK = 4096
M = 25 * TBLK


def _tc_transpose(tin_ref, o_ref):
    o_ref[:, 0:D] = tin_ref[...].T


def _transpose_table(table_t):
    return pl.pallas_call(
        _tc_transpose,
        grid=(M // TBLK,),
        in_specs=[pl.BlockSpec((D, TBLK), lambda i: (0, i))],
        out_specs=pl.BlockSpec((TBLK, 128), lambda i: (i, 0)),
        out_shape=jax.ShapeDtypeStruct((M, 128), jnp.float32),
    )(table_t)


@functools.partial(
    pl.kernel,
    out_type=jax.ShapeDtypeStruct((B, 128), jnp.float32),
    mesh=plsc.VectorSubcoreMesh(
        core_axis_name="c", subcore_axis_name="s",
        num_cores=NC, num_subcores=NS),
    scratch_types=[
        pltpu.VMEM((BPW, HIST), jnp.int32),
        pltpu.VMEM((HIST, D), jnp.float32),
        pltpu.VMEM((HIST, D), jnp.float32),
        pltpu.VMEM((BPW, 128), jnp.float32),
        pltpu.SemaphoreType.DMA,
        pltpu.SemaphoreType.DMA,
    ],
    compiler_params=pltpu.CompilerParams(use_tc_tiling_on_sc=False),
)
def _sc_hist_pool(hist_hbm, itab_hbm, hsum_out,
                  idx_v, rows_a, rows_b, hsum_v, sem_a, sem_b):
    w = lax.axis_index("s") * NC + lax.axis_index("c")
    base = w * BPW

    pltpu.sync_copy(hist_hbm.at[w], idx_v)

    def _accum(rows, r):
        def acc_body(j, acc):
            return tuple(acc[c] + rows[j, pl.ds(16 * c, 16)]
                         for c in range(4))

        z = jnp.zeros((16,), jnp.float32)
        acc = lax.fori_loop(0, HIST, acc_body, (z, z, z, z), unroll=5)
        for c in range(4):
            hsum_v[r, pl.ds(16 * c, 16)] = acc[c]

    pltpu.async_copy(itab_hbm.at[idx_v.at[0]], rows_a, sem_a)

    def pair_body(g, carry):
        r0 = 2 * g
        pltpu.async_copy(itab_hbm.at[idx_v.at[r0 + 1]], rows_b, sem_b)
        pltpu.make_async_copy(itab_hbm.at[idx_v.at[r0]], rows_a, sem_a).wait()
        _accum(rows_a, r0)

        @pl.when(r0 + 2 < BPW)
        def _():
            pltpu.async_copy(itab_hbm.at[idx_v.at[r0 + 2]], rows_a, sem_a)

        pltpu.make_async_copy(
            itab_hbm.at[idx_v.at[r0 + 1]], rows_b, sem_b).wait()
        _accum(rows_b, r0 + 1)
        return carry

    lax.fori_loop(0, BPW // 2, pair_body, 0)
    pltpu.sync_copy(hsum_v, hsum_out.at[pl.ds(base, BPW), :])


@functools.partial(
    pl.kernel,
    out_type=jax.ShapeDtypeStruct((B, 128), jnp.float32),
    mesh=plsc.VectorSubcoreMesh(
        core_axis_name="c", subcore_axis_name="s",
        num_cores=NC, num_subcores=NS),
    scratch_types=[
        pltpu.VMEM((BPW,), jnp.int32),
        pltpu.VMEM((BPW, 128), jnp.float32),
        pltpu.SemaphoreType.DMA,
    ],
    compiler_params=pltpu.CompilerParams(use_tc_tiling_on_sc=False),
)
def _sc_user_gather(uid_hbm, utab_hbm, u_out, uid_v, urows_v, sem):
    w = lax.axis_index("s") * NC + lax.axis_index("c")
    base = w * BPW
    pltpu.sync_copy(uid_hbm.at[pl.ds(base, BPW)], uid_v)
    pltpu.async_copy(utab_hbm.at[uid_v], urows_v, sem).wait()
    pltpu.sync_copy(urows_v, u_out.at[pl.ds(base, BPW), :])


def _tc_mlp(u_ref, hs_ref, hist_ref, tg_ref, cont_ref, gtab_ref,
            wc_ref, bc_ref, w1_ref, b1_ref, w2_ref, b2_ref, o_ref):
    hist = hist_ref[...]
    h_cnt = jnp.sum((hist > 0).astype(jnp.float32), axis=1, keepdims=True)
    hist_pool = hs_ref[:, 0:D] / (h_cnt + 1e-8)

    tg = tg_ref[...]
    gtab = gtab_ref[...]
    g_cnt = jnp.sum((tg > 0).astype(jnp.float32), axis=1, keepdims=True)
    g_sum = jnp.zeros_like(hist_pool)
    for v in range(1, GENRE_VOCAB):
        cv = jnp.sum((tg == v).astype(jnp.float32), axis=1, keepdims=True)
        g_sum = g_sum + cv * gtab[v:v + 1, :]
    g_pool = g_sum / (g_cnt + 1e-8)

    cont = cont_ref[...]
    wc = wc_ref[...]
    cont_emb = jnp.maximum(
        cont[:, 0:1] * wc[0:1, :] + cont[:, 1:2] * wc[1:2, :] + bc_ref[...],
        0.0)

    w1 = w1_ref[...]
    f32 = jnp.float32
    h = (jnp.dot(u_ref[:, 0:D], w1[0:64], preferred_element_type=f32)
         + jnp.dot(hist_pool, w1[64:128], preferred_element_type=f32)
         + jnp.dot(g_pool, w1[128:192], preferred_element_type=f32)
         + jnp.dot(cont_emb, w1[192:256], preferred_element_type=f32)
         + b1_ref[...])
    h = jnp.maximum(h, 0.0)
    out = jnp.dot(h, w2_ref[...], preferred_element_type=f32) + b2_ref[...]
    nrm = jnp.sqrt(jnp.sum(out * out, axis=1, keepdims=True))
    o_ref[...] = out / jnp.maximum(nrm, 1e-12)


def kernel(user_id, history, top_genres, avg_rating, activity,
           user_table, item_table, genre_table,
           W_cont, b_cont, W1, b1, W2, b2):
    uid = user_id.astype(jnp.int32)
    hist = history.astype(jnp.int32)
    hist3 = hist.reshape(NW, BPW, HIST)

    itabM = _transpose_table(item_table.T)
    itab2 = itabM.reshape(2 * M, D)
    hsum = _sc_hist_pool(hist3 * 2, itab2)
    utabM = _transpose_table(user_table.T)
    u_emb = _sc_user_gather(uid, utabM)

    cont = jnp.stack([avg_rating, activity], axis=1)

    bb = 512
    grid = (B // bb,)
    full = lambda shape: pl.BlockSpec(shape, lambda i: (0, 0))
    blk = lambda shape: pl.BlockSpec(shape, lambda i: (i, 0))

    out = pl.pallas_call(
        _tc_mlp,
        grid=grid,
        in_specs=[
            blk((bb, 128)),
            blk((bb, 128)),
            blk((bb, HIST)),
            blk((bb, NG)),
            blk((bb, 2)),
            full((GENRE_VOCAB, D)),
            full((2, D)),
            full((1, D)),
            full((4 * D, 128)),
            full((1, 128)),
            full((128, D)),
            full((1, D)),
        ],
        out_specs=blk((bb, D)),
        out_shape=jax.ShapeDtypeStruct((B, D), jnp.float32),
    )(u_emb, hsum, hist, top_genres.astype(jnp.int32), cont, genre_table,
      W_cont, b_cont.reshape(1, D), W1, b1.reshape(1, 128), W2,
      b2.reshape(1, D))
    return out


# speedup vs baseline: 8.8562x; 1.1083x over previous
import functools

import jax
import jax.numpy as jnp
from jax import lax
from jax.experimental import pallas as pl
from jax.experimental.pallas import tpu as pltpu
from jax.experimental.pallas import tpu_sc as plsc

B = 4096
HIST = 50
NG = 8
GENRE_VOCAB = 21
D = 64
VOCAB = 100001

NC = 2
NS = 16
NW = NC * NS
BPW = B // NW
NPAIR = BPW // 2
H2 = 2 * HIST

TBLK = 4096
M = 25 * TBLK


def _tc_transpose(tin_ref, o_ref):
    o_ref[:, 0:D] = tin_ref[...].T


def _transpose_table(table_t):
    return pl.pallas_call(
        _tc_transpose,
        grid=(M // TBLK,),
        in_specs=[pl.BlockSpec((D, TBLK), lambda i: (0, i))],
        out_specs=pl.BlockSpec((TBLK, 128), lambda i: (i, 0)),
        out_shape=jax.ShapeDtypeStruct((M, 128), jnp.float32),
    )(table_t)


@functools.partial(
    pl.kernel,
    out_type=jax.ShapeDtypeStruct((B, 128), jnp.float32),
    mesh=plsc.VectorSubcoreMesh(
        core_axis_name="c", subcore_axis_name="s",
        num_cores=NC, num_subcores=NS),
    scratch_types=[
        pltpu.VMEM((NPAIR, H2), jnp.int32),
        pltpu.VMEM((BPW, 16), jnp.int32),
        pltpu.VMEM((GENRE_VOCAB, D), jnp.float32),
        pltpu.VMEM((H2, D), jnp.float32),
        pltpu.VMEM((H2, D), jnp.float32),
        pltpu.VMEM((BPW, 128), jnp.float32),
        pltpu.SemaphoreType.DMA,
        pltpu.SemaphoreType.DMA,
    ],
    compiler_params=pltpu.CompilerParams(use_tc_tiling_on_sc=False),
)
def _sc_hist_pool(hist_hbm, tg_hbm, gtab_hbm, itab_hbm, hsum_out,
                  idx_v, tg_v, gtab_v, rows_a, rows_b, sum_v, sem_a, sem_b):
    w = lax.axis_index("s") * NC + lax.axis_index("c")
    base = w * BPW

    pltpu.sync_copy(hist_hbm.at[w], idx_v)
    pltpu.sync_copy(tg_hbm.at[w], tg_v)
    pltpu.sync_copy(gtab_hbm, gtab_v)

    def _accum(rows, off, r):
        def acc_body(j, acc):
            return tuple(acc[c] + rows[off + j, pl.ds(16 * c, 16)]
                         for c in range(4))

        z = jnp.zeros((16,), jnp.float32)
        acc = lax.fori_loop(0, HIST, acc_body, (z, z, z, z), unroll=5)
        for c in range(4):
            sum_v[r, pl.ds(16 * c, 16)] = acc[c]
        gacc = [jnp.zeros((16,), jnp.float32) for _ in range(4)]
        gv = tg_v[r, pl.ds(0, 16)]
        for j in range(NG):
            gid = gv[j]
            for c in range(4):
                gacc[c] = gacc[c] + gtab_v[gid, pl.ds(16 * c, 16)]
        for c in range(4):
            sum_v[r, pl.ds(D + 16 * c, 16)] = gacc[c]

    pltpu.async_copy(itab_hbm.at[idx_v.at[0]], rows_a, sem_a)

    def quad_body(g, carry):
        p0 = 2 * g
        pltpu.async_copy(itab_hbm.at[idx_v.at[p0 + 1]], rows_b, sem_b)
        pltpu.make_async_copy(itab_hbm.at[idx_v.at[p0]], rows_a, sem_a).wait()
        _accum(rows_a, 0, 2 * p0)
        _accum(rows_a, HIST, 2 * p0 + 1)

        @pl.when(p0 + 2 < NPAIR)
        def _():
            pltpu.async_copy(itab_hbm.at[idx_v.at[p0 + 2]], rows_a, sem_a)

        pltpu.make_async_copy(
            itab_hbm.at[idx_v.at[p0 + 1]], rows_b, sem_b).wait()
        _accum(rows_b, 0, 2 * p0 + 2)
        _accum(rows_b, HIST, 2 * p0 + 3)
        return carry

    lax.fori_loop(0, NPAIR // 2, quad_body, 0)
    pltpu.sync_copy(sum_v, hsum_out.at[pl.ds(base, BPW), :])


@functools.partial(
    pl.kernel,
    out_type=jax.ShapeDtypeStruct((B, 128), jnp.float32),
    mesh=plsc.VectorSubcoreMesh(
        core_axis_name="c", subcore_axis_name="s",
        num_cores=NC, num_subcores=NS),
    scratch_types=[
        pltpu.VMEM((BPW,), jnp.int32),
        pltpu.VMEM((BPW, 128), jnp.float32),
        pltpu.SemaphoreType.DMA,
    ],
    compiler_params=pltpu.CompilerParams(use_tc_tiling_on_sc=False),
)
def _sc_user_gather(uid_hbm, utab_hbm, u_out, uid_v, urows_v, sem):
    w = lax.axis_index("s") * NC + lax.axis_index("c")
    base = w * BPW
    pltpu.sync_copy(uid_hbm.at[pl.ds(base, BPW)], uid_v)
    pltpu.async_copy(utab_hbm.at[uid_v], urows_v, sem).wait()
    pltpu.sync_copy(urows_v, u_out.at[pl.ds(base, BPW), :])


def _tc_mlp(u_ref, hs_ref, hist_ref, tg_ref, cont_ref,
            wc_ref, bc_ref, w1_ref, b1_ref, w2_ref, b2_ref, o_ref):
    hist = hist_ref[...]
    h_cnt = jnp.sum((hist > 0).astype(jnp.float32), axis=1, keepdims=True)
    hist_pool = hs_ref[:, 0:D] / (h_cnt + 1e-8)

    tg = tg_ref[...]
    g_cnt = jnp.sum((tg > 0).astype(jnp.float32), axis=1, keepdims=True)
    g_pool = hs_ref[:, D:2 * D] / (g_cnt + 1e-8)

    cont = cont_ref[...]
    wc = wc_ref[...]
    cont_emb = jnp.maximum(
        cont[:, 0:1] * wc[0:1, :] + cont[:, 1:2] * wc[1:2, :] + bc_ref[...],
        0.0)

    w1 = w1_ref[...]
    f32 = jnp.float32
    h = (jnp.dot(u_ref[:, 0:D], w1[0:64], preferred_element_type=f32)
         + jnp.dot(hist_pool, w1[64:128], preferred_element_type=f32)
         + jnp.dot(g_pool, w1[128:192], preferred_element_type=f32)
         + jnp.dot(cont_emb, w1[192:256], preferred_element_type=f32)
         + b1_ref[...])
    h = jnp.maximum(h, 0.0)
    out = jnp.dot(h, w2_ref[...], preferred_element_type=f32) + b2_ref[...]
    nrm = jnp.sqrt(jnp.sum(out * out, axis=1, keepdims=True))
    o_ref[...] = out / jnp.maximum(nrm, 1e-12)


def kernel(user_id, history, top_genres, avg_rating, activity,
           user_table, item_table, genre_table,
           W_cont, b_cont, W1, b1, W2, b2):
    uid = user_id.astype(jnp.int32)
    hist = history.astype(jnp.int32)
    tg = top_genres.astype(jnp.int32)
    hist3 = (hist * 2).reshape(NW, NPAIR, H2)
    tg3 = jnp.pad(tg, ((0, 0), (0, 16 - NG))).reshape(NW, BPW, 16)

    itabM = _transpose_table(item_table.T)
    itab2 = itabM.reshape(2 * M, D)
    hsum = _sc_hist_pool(hist3, tg3, genre_table, itab2)
    utabM = _transpose_table(user_table.T)
    u_emb = _sc_user_gather(uid, utabM)

    cont = jnp.stack([avg_rating, activity], axis=1)

    bb = 512
    grid = (B // bb,)
    full = lambda shape: pl.BlockSpec(shape, lambda i: (0, 0))
    blk = lambda shape: pl.BlockSpec(shape, lambda i: (i, 0))

    out = pl.pallas_call(
        _tc_mlp,
        grid=grid,
        in_specs=[
            blk((bb, 128)),
            blk((bb, 128)),
            blk((bb, HIST)),
            blk((bb, NG)),
            blk((bb, 2)),
            full((2, D)),
            full((1, D)),
            full((4 * D, 128)),
            full((1, 128)),
            full((128, D)),
            full((1, D)),
        ],
        out_specs=blk((bb, D)),
        out_shape=jax.ShapeDtypeStruct((B, D), jnp.float32),
    )(u_emb, hsum, hist, tg, cont,
      W_cont, b_cont.reshape(1, D), W1, b1.reshape(1, 128), W2,
      b2.reshape(1, D))
    return out


# speedup vs baseline: 9.3047x; 1.0507x over previous
import functools

import jax
import jax.numpy as jnp
from jax import lax
from jax.experimental import pallas as pl
from jax.experimental.pallas import tpu as pltpu
from jax.experimental.pallas import tpu_sc as plsc

B = 4096
HIST = 50
NG = 8
GENRE_VOCAB = 21
D = 64
VOCAB = 100001

NC = 2
NS = 16
NW = NC * NS
BPW = B // NW
NPAIR = BPW // 2
H2 = 2 * HIST

TBLK = 8192
M = 13 * TBLK


def _tc_transpose(tin_ref, o_ref):
    o_ref[:, 0:D] = tin_ref[...].T


def _transpose_table(table_t):
    return pl.pallas_call(
        _tc_transpose,
        grid=(M // TBLK,),
        in_specs=[pl.BlockSpec((D, TBLK), lambda i: (0, i))],
        out_specs=pl.BlockSpec((TBLK, 128), lambda i: (i, 0)),
        out_shape=jax.ShapeDtypeStruct((M, 128), jnp.float32),
    )(table_t)


@functools.partial(
    pl.kernel,
    out_type=jax.ShapeDtypeStruct((B, 128), jnp.float32),
    mesh=plsc.VectorSubcoreMesh(
        core_axis_name="c", subcore_axis_name="s",
        num_cores=NC, num_subcores=NS),
    scratch_types=[
        pltpu.VMEM((NPAIR, H2), jnp.int32),
        pltpu.VMEM((BPW, 16), jnp.int32),
        pltpu.VMEM((GENRE_VOCAB, D), jnp.float32),
        pltpu.VMEM((H2, D), jnp.float32),
        pltpu.VMEM((H2, D), jnp.float32),
        pltpu.VMEM((BPW, 128), jnp.float32),
        pltpu.SemaphoreType.DMA,
        pltpu.SemaphoreType.DMA,
    ],
    compiler_params=pltpu.CompilerParams(use_tc_tiling_on_sc=False),
)
def _sc_hist_pool(hist_hbm, tg_hbm, gtab_hbm, itab_hbm, hsum_out,
                  idx_v, tg_v, gtab_v, rows_a, rows_b, sum_v, sem_a, sem_b):
    w = lax.axis_index("s") * NC + lax.axis_index("c")
    base = w * BPW

    pltpu.sync_copy(hist_hbm.at[w], idx_v)
    pltpu.sync_copy(tg_hbm.at[w], tg_v)
    pltpu.sync_copy(gtab_hbm, gtab_v)

    def _accum(rows, off, r):
        def acc_body(j, acc):
            return tuple(acc[c] + rows[off + j, pl.ds(16 * c, 16)]
                         for c in range(4))

        z = jnp.zeros((16,), jnp.float32)
        acc = lax.fori_loop(0, HIST, acc_body, (z, z, z, z), unroll=5)
        for c in range(4):
            sum_v[r, pl.ds(16 * c, 16)] = acc[c]
        gacc = [jnp.zeros((16,), jnp.float32) for _ in range(4)]
        gv = tg_v[r, pl.ds(0, 16)]
        for j in range(NG):
            gid = gv[j]
            for c in range(4):
                gacc[c] = gacc[c] + gtab_v[gid, pl.ds(16 * c, 16)]
        for c in range(4):
            sum_v[r, pl.ds(D + 16 * c, 16)] = gacc[c]

    pltpu.async_copy(itab_hbm.at[idx_v.at[0]], rows_a, sem_a)

    def quad_body(g, carry):
        p0 = 2 * g
        pltpu.async_copy(itab_hbm.at[idx_v.at[p0 + 1]], rows_b, sem_b)
        pltpu.make_async_copy(itab_hbm.at[idx_v.at[p0]], rows_a, sem_a).wait()
        _accum(rows_a, 0, 2 * p0)
        _accum(rows_a, HIST, 2 * p0 + 1)

        @pl.when(p0 + 2 < NPAIR)
        def _():
            pltpu.async_copy(itab_hbm.at[idx_v.at[p0 + 2]], rows_a, sem_a)

        pltpu.make_async_copy(
            itab_hbm.at[idx_v.at[p0 + 1]], rows_b, sem_b).wait()
        _accum(rows_b, 0, 2 * p0 + 2)
        _accum(rows_b, HIST, 2 * p0 + 3)
        return carry

    lax.fori_loop(0, NPAIR // 2, quad_body, 0)
    pltpu.sync_copy(sum_v, hsum_out.at[pl.ds(base, BPW), :])


@functools.partial(
    pl.kernel,
    out_type=jax.ShapeDtypeStruct((B, 128), jnp.float32),
    mesh=plsc.VectorSubcoreMesh(
        core_axis_name="c", subcore_axis_name="s",
        num_cores=NC, num_subcores=NS),
    scratch_types=[
        pltpu.VMEM((BPW,), jnp.int32),
        pltpu.VMEM((BPW, 128), jnp.float32),
        pltpu.SemaphoreType.DMA,
    ],
    compiler_params=pltpu.CompilerParams(use_tc_tiling_on_sc=False),
)
def _sc_user_gather(uid_hbm, utab_hbm, u_out, uid_v, urows_v, sem):
    w = lax.axis_index("s") * NC + lax.axis_index("c")
    base = w * BPW
    pltpu.sync_copy(uid_hbm.at[pl.ds(base, BPW)], uid_v)
    pltpu.async_copy(utab_hbm.at[uid_v], urows_v, sem).wait()
    pltpu.sync_copy(urows_v, u_out.at[pl.ds(base, BPW), :])


def _tc_mlp(u_ref, hs_ref, hist_ref, tg_ref, cont_ref,
            wc_ref, bc_ref, w1_ref, b1_ref, w2_ref, b2_ref, o_ref):
    hist = hist_ref[...]
    h_cnt = jnp.sum((hist > 0).astype(jnp.float32), axis=1, keepdims=True)
    hist_pool = hs_ref[:, 0:D] / (h_cnt + 1e-8)

    tg = tg_ref[...]
    g_cnt = jnp.sum((tg > 0).astype(jnp.float32), axis=1, keepdims=True)
    g_pool = hs_ref[:, D:2 * D] / (g_cnt + 1e-8)

    cont = cont_ref[...]
    wc = wc_ref[...]
    cont_emb = jnp.maximum(
        cont[:, 0:1] * wc[0:1, :] + cont[:, 1:2] * wc[1:2, :] + bc_ref[...],
        0.0)

    w1 = w1_ref[...]
    f32 = jnp.float32
    h = (jnp.dot(u_ref[:, 0:D], w1[0:64], preferred_element_type=f32)
         + jnp.dot(hist_pool, w1[64:128], preferred_element_type=f32)
         + jnp.dot(g_pool, w1[128:192], preferred_element_type=f32)
         + jnp.dot(cont_emb, w1[192:256], preferred_element_type=f32)
         + b1_ref[...])
    h = jnp.maximum(h, 0.0)
    out = jnp.dot(h, w2_ref[...], preferred_element_type=f32) + b2_ref[...]
    nrm = jnp.sqrt(jnp.sum(out * out, axis=1, keepdims=True))
    o_ref[...] = out / jnp.maximum(nrm, 1e-12)


def kernel(user_id, history, top_genres, avg_rating, activity,
           user_table, item_table, genre_table,
           W_cont, b_cont, W1, b1, W2, b2):
    uid = user_id.astype(jnp.int32)
    hist = history.astype(jnp.int32)
    tg = top_genres.astype(jnp.int32)
    hist3 = (hist * 2).reshape(NW, NPAIR, H2)
    tg3 = jnp.pad(tg, ((0, 0), (0, 16 - NG))).reshape(NW, BPW, 16)

    utabM = _transpose_table(user_table.T)
    u_emb = _sc_user_gather(uid, utabM)
    itabM = _transpose_table(item_table.T)
    itab2 = itabM.reshape(2 * M, D)
    hsum = _sc_hist_pool(hist3, tg3, genre_table, itab2)

    cont = jnp.stack([avg_rating, activity], axis=1)

    bb = 1024
    grid = (B // bb,)
    full = lambda shape: pl.BlockSpec(shape, lambda i: (0, 0))
    blk = lambda shape: pl.BlockSpec(shape, lambda i: (i, 0))

    out = pl.pallas_call(
        _tc_mlp,
        grid=grid,
        in_specs=[
            blk((bb, 128)),
            blk((bb, 128)),
            blk((bb, HIST)),
            blk((bb, NG)),
            blk((bb, 2)),
            full((2, D)),
            full((1, D)),
            full((4 * D, 128)),
            full((1, 128)),
            full((128, D)),
            full((1, D)),
        ],
        out_specs=blk((bb, D)),
        out_shape=jax.ShapeDtypeStruct((B, D), jnp.float32),
    )(u_emb, hsum, hist, tg, cont,
      W_cont, b_cont.reshape(1, D), W1, b1.reshape(1, 128), W2,
      b2.reshape(1, D))
    return out


# speedup vs baseline: 9.5159x; 1.0227x over previous
import functools

import jax
import jax.numpy as jnp
from jax import lax
from jax.experimental import pallas as pl
from jax.experimental.pallas import tpu as pltpu
from jax.experimental.pallas import tpu_sc as plsc

B = 4096
HIST = 50
NG = 8
GENRE_VOCAB = 21
D = 64
VOCAB = 100001

NC = 2
NS = 16
NW = NC * NS
BPW = B // NW
NPAIR = BPW // 2
H2 = 2 * HIST

TBLK = 8192
M = 13 * TBLK


def _tc_transpose(tin_ref, o_ref):
    o_ref[:, 0:D] = tin_ref[...].T


def _transpose_table(table_t):
    return pl.pallas_call(
        _tc_transpose,
        grid=(M // TBLK,),
        in_specs=[pl.BlockSpec((D, TBLK), lambda i: (0, i))],
        out_specs=pl.BlockSpec((TBLK, 128), lambda i: (i, 0)),
        out_shape=jax.ShapeDtypeStruct((M, 128), jnp.float32),
    )(table_t)


@functools.partial(
    pl.kernel,
    out_type=jax.ShapeDtypeStruct((B, 128), jnp.float32),
    mesh=plsc.VectorSubcoreMesh(
        core_axis_name="c", subcore_axis_name="s",
        num_cores=NC, num_subcores=NS),
    scratch_types=[
        pltpu.VMEM((NPAIR, H2), jnp.int32),
        pltpu.VMEM((BPW, 16), jnp.int32),
        pltpu.VMEM((GENRE_VOCAB, D), jnp.float32),
        pltpu.VMEM((H2, D), jnp.float32),
        pltpu.VMEM((H2, D), jnp.float32),
        pltpu.VMEM((BPW, 128), jnp.float32),
        pltpu.SemaphoreType.DMA,
        pltpu.SemaphoreType.DMA,
    ],
    compiler_params=pltpu.CompilerParams(
        use_tc_tiling_on_sc=False, needs_layout_passes=False),
)
def _sc_hist_pool(hist_hbm, tg_hbm, gtab_hbm, itab_hbm, hsum_out,
                  idx_v, tg_v, gtab_v, rows_a, rows_b, sum_v, sem_a, sem_b):
    w = lax.axis_index("s") * NC + lax.axis_index("c")
    base = w * BPW

    pltpu.sync_copy(hist_hbm.at[w], idx_v)
    pltpu.sync_copy(tg_hbm.at[w], tg_v)
    pltpu.sync_copy(gtab_hbm, gtab_v)

    def _accum(rows, off, r, p):
        def acc_body(j, acc):
            return tuple(acc[c] + rows[off + j, pl.ds(16 * c, 16)]
                         for c in range(4))

        z = jnp.zeros((16,), jnp.float32)
        acc = lax.fori_loop(0, HIST, acc_body, (z, z, z, z), unroll=5)
        ones = jnp.ones((16,), jnp.float32)
        zs = jnp.zeros((16,), jnp.float32)
        lane = lax.iota(jnp.int32, 16)
        i0 = idx_v[p, pl.ds(off, 16)]
        i1 = idx_v[p, pl.ds(off + 16, 16)]
        i2 = idx_v[p, pl.ds(off + 32, 16)]
        i3 = idx_v[p, pl.ds(off + 34, 16)]
        cs = (jnp.where(i0 > 0, ones, zs) + jnp.where(i1 > 0, ones, zs)
              + jnp.where(i2 > 0, ones, zs)
              + jnp.where((i3 > 0) & (lane >= 14), ones, zs))
        hden = jnp.sum(cs) + 1e-8
        for c in range(4):
            sum_v[r, pl.ds(16 * c, 16)] = acc[c] / hden
        gacc = [jnp.zeros((16,), jnp.float32) for _ in range(4)]
        gv = tg_v[r, pl.ds(0, 16)]
        for j in range(NG):
            gid = gv[j]
            for c in range(4):
                gacc[c] = gacc[c] + gtab_v[gid, pl.ds(16 * c, 16)]
        gden = jnp.sum(jnp.where(gv > 0, ones, zs)) + 1e-8
        for c in range(4):
            sum_v[r, pl.ds(D + 16 * c, 16)] = gacc[c] / gden

    pltpu.async_copy(itab_hbm.at[idx_v.at[0]], rows_a, sem_a)

    def quad_body(g, carry):
        p0 = 2 * g
        pltpu.async_copy(itab_hbm.at[idx_v.at[p0 + 1]], rows_b, sem_b)
        pltpu.make_async_copy(itab_hbm.at[idx_v.at[p0]], rows_a, sem_a).wait()
        _accum(rows_a, 0, 2 * p0, p0)
        _accum(rows_a, HIST, 2 * p0 + 1, p0)

        @pl.when(p0 + 2 < NPAIR)
        def _():
            pltpu.async_copy(itab_hbm.at[idx_v.at[p0 + 2]], rows_a, sem_a)

        pltpu.make_async_copy(
            itab_hbm.at[idx_v.at[p0 + 1]], rows_b, sem_b).wait()
        _accum(rows_b, 0, 2 * p0 + 2, p0 + 1)
        _accum(rows_b, HIST, 2 * p0 + 3, p0 + 1)
        return carry

    lax.fori_loop(0, NPAIR // 2, quad_body, 0)
    pltpu.sync_copy(sum_v, hsum_out.at[pl.ds(base, BPW), :])


@functools.partial(
    pl.kernel,
    out_type=jax.ShapeDtypeStruct((B, 128), jnp.float32),
    mesh=plsc.VectorSubcoreMesh(
        core_axis_name="c", subcore_axis_name="s",
        num_cores=NC, num_subcores=NS),
    scratch_types=[
        pltpu.VMEM((BPW,), jnp.int32),
        pltpu.VMEM((BPW, 128), jnp.float32),
        pltpu.SemaphoreType.DMA,
    ],
    compiler_params=pltpu.CompilerParams(use_tc_tiling_on_sc=False),
)
def _sc_user_gather(uid_hbm, utab_hbm, u_out, uid_v, urows_v, sem):
    w = lax.axis_index("s") * NC + lax.axis_index("c")
    base = w * BPW
    pltpu.sync_copy(uid_hbm.at[pl.ds(base, BPW)], uid_v)
    pltpu.async_copy(utab_hbm.at[uid_v], urows_v, sem).wait()
    pltpu.sync_copy(urows_v, u_out.at[pl.ds(base, BPW), :])


def _tc_mlp(u_ref, hs_ref, cont_ref,
            wc_ref, bc_ref, w1_ref, b1_ref, w2_ref, b2_ref, o_ref):
    hist_pool = hs_ref[:, 0:D]
    g_pool = hs_ref[:, D:2 * D]

    cont = cont_ref[...]
    wc = wc_ref[...]
    cont_emb = jnp.maximum(
        cont[:, 0:1] * wc[0:1, :] + cont[:, 1:2] * wc[1:2, :] + bc_ref[...],
        0.0)

    w1 = w1_ref[...]
    f32 = jnp.float32
    h = (jnp.dot(u_ref[:, 0:D], w1[0:64], preferred_element_type=f32)
         + jnp.dot(hist_pool, w1[64:128], preferred_element_type=f32)
         + jnp.dot(g_pool, w1[128:192], preferred_element_type=f32)
         + jnp.dot(cont_emb, w1[192:256], preferred_element_type=f32)
         + b1_ref[...])
    h = jnp.maximum(h, 0.0)
    out = jnp.dot(h, w2_ref[...], preferred_element_type=f32) + b2_ref[...]
    nrm = jnp.sqrt(jnp.sum(out * out, axis=1, keepdims=True))
    o_ref[...] = out / jnp.maximum(nrm, 1e-12)


def kernel(user_id, history, top_genres, avg_rating, activity,
           user_table, item_table, genre_table,
           W_cont, b_cont, W1, b1, W2, b2):
    uid = user_id.astype(jnp.int32)
    hist = history.astype(jnp.int32)
    tg = top_genres.astype(jnp.int32)
    hist3 = (hist * 2).reshape(NW, NPAIR, H2)
    tg3 = jnp.pad(tg, ((0, 0), (0, 16 - NG))).reshape(NW, BPW, 16)

    utabM = _transpose_table(user_table.T)
    u_emb = _sc_user_gather(uid, utabM)
    itabM = _transpose_table(item_table.T)
    itab2 = itabM.reshape(2 * M, D)
    hsum = _sc_hist_pool(hist3, tg3, genre_table, itab2)

    cont = jnp.stack([avg_rating, activity], axis=1)

    bb = 1024
    grid = (B // bb,)
    full = lambda shape: pl.BlockSpec(shape, lambda i: (0, 0))
    blk = lambda shape: pl.BlockSpec(shape, lambda i: (i, 0))

    out = pl.pallas_call(
        _tc_mlp,
        grid=grid,
        in_specs=[
            blk((bb, 128)),
            blk((bb, 128)),
            blk((bb, 2)),
            full((2, D)),
            full((1, D)),
            full((4 * D, 128)),
            full((1, 128)),
            full((128, D)),
            full((1, D)),
        ],
        out_specs=blk((bb, D)),
        out_shape=jax.ShapeDtypeStruct((B, D), jnp.float32),
    )(u_emb, hsum, cont,
      W_cont, b_cont.reshape(1, D), W1, b1.reshape(1, 128), W2,
      b2.reshape(1, D))
    return out


# speedup vs baseline: 9.7526x; 1.0249x over previous
import functools

import jax
import jax.numpy as jnp
from jax import lax
from jax.experimental import pallas as pl
from jax.experimental.pallas import tpu as pltpu
from jax.experimental.pallas import tpu_sc as plsc

B = 4096
HIST = 50
NG = 8
GENRE_VOCAB = 21
D = 64
VOCAB = 100001

NC = 2
NS = 16
NW = NC * NS
BPW = B // NW
NPAIR = BPW // 2
H2 = 2 * HIST

TBLK = 8192
M = 13 * TBLK


def _tc_transpose(tin_ref, o_ref):
    o_ref[:, 0:D] = tin_ref[...].T


def _transpose_table(table_t):
    return pl.pallas_call(
        _tc_transpose,
        grid=(M // TBLK,),
        in_specs=[pl.BlockSpec((D, TBLK), lambda i: (0, i))],
        out_specs=pl.BlockSpec((TBLK, 128), lambda i: (i, 0)),
        out_shape=jax.ShapeDtypeStruct((M, 128), jnp.float32),
    )(table_t)


@functools.partial(
    pl.kernel,
    out_type=jax.ShapeDtypeStruct((B, 128), jnp.float32),
    mesh=plsc.VectorSubcoreMesh(
        core_axis_name="c", subcore_axis_name="s",
        num_cores=NC, num_subcores=NS),
    scratch_types=[
        pltpu.VMEM((NPAIR, H2), jnp.int32),
        pltpu.VMEM((BPW, 16), jnp.int32),
        pltpu.VMEM((GENRE_VOCAB, D), jnp.float32),
        pltpu.VMEM((H2, D), jnp.float32),
        pltpu.VMEM((H2, D), jnp.float32),
        pltpu.VMEM((BPW, 128), jnp.float32),
        pltpu.SemaphoreType.DMA,
        pltpu.SemaphoreType.DMA,
    ],
    compiler_params=pltpu.CompilerParams(
        use_tc_tiling_on_sc=False, needs_layout_passes=False),
)
def _sc_hist_pool(hist_hbm, tg_hbm, gtab_hbm, itab_hbm, hsum_out,
                  idx_v, tg_v, gtab_v, rows_a, rows_b, sum_v, sem_a, sem_b):
    w = lax.axis_index("s") * NC + lax.axis_index("c")
    base = w * BPW

    pltpu.sync_copy(hist_hbm.at[w], idx_v)
    pltpu.sync_copy(tg_hbm.at[w], tg_v)
    pltpu.sync_copy(gtab_hbm, gtab_v)

    def _accum(rows, off, r, p):
        def acc_body(j, acc):
            return tuple(acc[c] + rows[off + j, pl.ds(16 * c, 16)]
                         for c in range(4))

        z = jnp.zeros((16,), jnp.float32)
        acc = lax.fori_loop(0, HIST, acc_body, (z, z, z, z), unroll=5)
        ones = jnp.ones((16,), jnp.float32)
        zs = jnp.zeros((16,), jnp.float32)
        lane = lax.iota(jnp.int32, 16)
        i0 = idx_v[p, pl.ds(off, 16)]
        i1 = idx_v[p, pl.ds(off + 16, 16)]
        i2 = idx_v[p, pl.ds(off + 32, 16)]
        i3 = idx_v[p, pl.ds(off + 34, 16)]
        cs = (jnp.where(i0 > 0, ones, zs) + jnp.where(i1 > 0, ones, zs)
              + jnp.where(i2 > 0, ones, zs)
              + jnp.where((i3 > 0) & (lane >= 14), ones, zs))
        hden = jnp.sum(cs) + 1e-8
        for c in range(4):
            sum_v[r, pl.ds(16 * c, 16)] = acc[c] / hden
        gacc = [jnp.zeros((16,), jnp.float32) for _ in range(4)]
        gv = tg_v[r, pl.ds(0, 16)]
        for j in range(NG):
            gid = gv[j]
            for c in range(4):
                gacc[c] = gacc[c] + gtab_v[gid, pl.ds(16 * c, 16)]
        gden = jnp.sum(jnp.where(gv > 0, ones, zs)) + 1e-8
        for c in range(4):
            sum_v[r, pl.ds(D + 16 * c, 16)] = gacc[c] / gden

    pltpu.async_copy(itab_hbm.at[idx_v.at[0]], rows_a, sem_a)

    def quad_body(g, carry):
        p0 = 2 * g
        pltpu.async_copy(itab_hbm.at[idx_v.at[p0 + 1]], rows_b, sem_b)
        pltpu.make_async_copy(itab_hbm.at[idx_v.at[p0]], rows_a, sem_a).wait()
        _accum(rows_a, 0, 2 * p0, p0)
        _accum(rows_a, HIST, 2 * p0 + 1, p0)

        @pl.when(p0 + 2 < NPAIR)
        def _():
            pltpu.async_copy(itab_hbm.at[idx_v.at[p0 + 2]], rows_a, sem_a)

        pltpu.make_async_copy(
            itab_hbm.at[idx_v.at[p0 + 1]], rows_b, sem_b).wait()
        _accum(rows_b, 0, 2 * p0 + 2, p0 + 1)
        _accum(rows_b, HIST, 2 * p0 + 3, p0 + 1)
        return carry

    lax.fori_loop(0, NPAIR // 2, quad_body, 0)
    pltpu.sync_copy(sum_v, hsum_out.at[pl.ds(base, BPW), :])


@functools.partial(
    pl.kernel,
    out_type=jax.ShapeDtypeStruct((B, 128), jnp.float32),
    mesh=plsc.VectorSubcoreMesh(
        core_axis_name="c", subcore_axis_name="s",
        num_cores=NC, num_subcores=NS),
    scratch_types=[
        pltpu.VMEM((BPW,), jnp.int32),
        pltpu.VMEM((BPW, 128), jnp.float32),
        pltpu.SemaphoreType.DMA,
    ],
    compiler_params=pltpu.CompilerParams(use_tc_tiling_on_sc=False),
)
def _sc_user_gather(uid_hbm, utab_hbm, u_out, uid_v, urows_v, sem):
    w = lax.axis_index("s") * NC + lax.axis_index("c")
    base = w * BPW
    pltpu.sync_copy(uid_hbm.at[pl.ds(base, BPW)], uid_v)
    pltpu.async_copy(utab_hbm.at[uid_v], urows_v, sem).wait()
    pltpu.sync_copy(urows_v, u_out.at[pl.ds(base, BPW), :])


def _tc_mlp(u_ref, hs_ref, cont_ref,
            wc_ref, bc_ref, w1_ref, b1_ref, w2_ref, b2_ref, o_ref):
    hist_pool = hs_ref[:, 0:D]
    g_pool = hs_ref[:, D:2 * D]

    cont = cont_ref[...]
    wc = wc_ref[...]
    cont_emb = jnp.maximum(
        cont[:, 0:1] * wc[0:1, :] + cont[:, 1:2] * wc[1:2, :] + bc_ref[...],
        0.0)

    w1 = w1_ref[...]
    f32 = jnp.float32
    h = (jnp.dot(u_ref[:, 0:D], w1[0:64], preferred_element_type=f32)
         + jnp.dot(hist_pool, w1[64:128], preferred_element_type=f32)
         + jnp.dot(g_pool, w1[128:192], preferred_element_type=f32)
         + jnp.dot(cont_emb, w1[192:256], preferred_element_type=f32)
         + b1_ref[...])
    h = jnp.maximum(h, 0.0)
    out = jnp.dot(h, w2_ref[...], preferred_element_type=f32) + b2_ref[...]
    nrm = jnp.sqrt(jnp.sum(out * out, axis=1, keepdims=True))
    o_ref[...] = (out / jnp.maximum(nrm, 1e-12)).T


def kernel(user_id, history, top_genres, avg_rating, activity,
           user_table, item_table, genre_table,
           W_cont, b_cont, W1, b1, W2, b2):
    uid = user_id.astype(jnp.int32)
    hist = history.astype(jnp.int32)
    tg = top_genres.astype(jnp.int32)
    hist3 = (hist * 2).reshape(NW, NPAIR, H2)
    tg3 = jnp.pad(tg, ((0, 0), (0, 16 - NG))).reshape(NW, BPW, 16)

    utabM = _transpose_table(user_table.T)
    u_emb = _sc_user_gather(uid, utabM)
    itabM = _transpose_table(item_table.T)
    itab2 = itabM.reshape(2 * M, D)
    hsum = _sc_hist_pool(hist3, tg3, genre_table, itab2)

    cont = jnp.stack([avg_rating, activity], axis=1)

    bb = 2048
    grid = (B // bb,)
    full = lambda shape: pl.BlockSpec(shape, lambda i: (0, 0))
    blk = lambda shape: pl.BlockSpec(shape, lambda i: (i, 0))

    out = pl.pallas_call(
        _tc_mlp,
        grid=grid,
        in_specs=[
            blk((bb, 128)),
            blk((bb, 128)),
            blk((bb, 2)),
            full((2, D)),
            full((1, D)),
            full((4 * D, 128)),
            full((1, 128)),
            full((128, D)),
            full((1, D)),
        ],
        out_specs=pl.BlockSpec((D, bb), lambda i: (0, i)),
        out_shape=jax.ShapeDtypeStruct((D, B), jnp.float32),
    )(u_emb, hsum, cont,
      W_cont, b_cont.reshape(1, D), W1, b1.reshape(1, 128), W2,
      b2.reshape(1, D))
    return out.T
